# baseline (speedup 1.0000x reference)
import functools
import math

import jax
import jax.numpy as jnp
from jax import lax
from jax.experimental import pallas as pl
from jax.experimental.pallas import tpu as pltpu

N = 50000
E = 800000
D = 64
AVG_DEG_LOG = math.log(17.0)

NB = 1000
EB = 2000
GN = N // NB
GE = E // EB

_HIGH = jax.lax.Precision.HIGHEST


def _dot(a, b):
    return jax.lax.dot_general(a, b, (((1,), (0,)), ((), ())),
                               precision=_HIGH,
                               preferred_element_type=jnp.float32)


def _prep_body(x_ref, wpre_ref, we_ref, be_ref, bpre_ref,
               xw1_ref, xw2_ref, wea_ref, cea_ref):
    w1 = wpre_ref[0:D, :]
    w2 = wpre_ref[D:2 * D, :]
    w3 = wpre_ref[2 * D:3 * D, :]
    xw1_ref[...] = _dot(x_ref[...], w1)
    xw2_ref[...] = _dot(x_ref[...], w2)
    wea_ref[...] = _dot(we_ref[...], w3)
    cea_ref[...] = _dot(be_ref[...], w3) + bpre_ref[...]


def _prep(x, Wpre, We, be, bpre):
    return pl.pallas_call(
        _prep_body,
        grid=(GN,),
        in_specs=[
            pl.BlockSpec((NB, D), lambda i: (i, 0)),
            pl.BlockSpec((3 * D, D), lambda i: (0, 0)),
            pl.BlockSpec((D, D), lambda i: (0, 0)),
            pl.BlockSpec((1, D), lambda i: (0, 0)),
            pl.BlockSpec((1, D), lambda i: (0, 0)),
        ],
        out_specs=[
            pl.BlockSpec((NB, D), lambda i: (i, 0)),
            pl.BlockSpec((NB, D), lambda i: (i, 0)),
            pl.BlockSpec((D, D), lambda i: (0, 0)),
            pl.BlockSpec((1, D), lambda i: (0, 0)),
        ],
        out_shape=[
            jax.ShapeDtypeStruct((N, D), jnp.float32),
            jax.ShapeDtypeStruct((N, D), jnp.float32),
            jax.ShapeDtypeStruct((D, D), jnp.float32),
            jax.ShapeDtypeStruct((1, D), jnp.float32),
        ],
    )(x, Wpre, We, be.reshape(1, D), bpre.reshape(1, D))


def _edge_m_body(g_ref, ea_ref, wea_ref, cea_ref, m_ref):
    m_ref[...] = g_ref[...] + _dot(ea_ref[...], wea_ref[...]) + cea_ref[...]


def _edge_m(g, ea, Wea, cea):
    return pl.pallas_call(
        _edge_m_body,
        grid=(GE,),
        in_specs=[
            pl.BlockSpec((EB, D), lambda i: (i, 0)),
            pl.BlockSpec((EB, D), lambda i: (i, 0)),
            pl.BlockSpec((D, D), lambda i: (0, 0)),
            pl.BlockSpec((1, D), lambda i: (0, 0)),
        ],
        out_specs=pl.BlockSpec((EB, D), lambda i: (i, 0)),
        out_shape=jax.ShapeDtypeStruct((E, D), jnp.float32),
    )(g, ea, Wea, cea)


def _node_body(sum_ref, sq_ref, mx_ref, mn_ref, cnt_ref, x_ref,
               wpost_ref, bpost_ref, wlin_ref, blin_ref,
               conv_ref, parts_ref):
    cnt = cnt_ref[...]
    deg = jnp.maximum(cnt, 1.0)
    inv = 1.0 / deg
    mean = sum_ref[...] * inv
    mean2 = sq_ref[...] * inv
    std = jnp.sqrt(jnp.maximum(mean2 - mean * mean, 0.0) + 1e-5)
    has = cnt > 0.0
    mx = jnp.where(has, mx_ref[...], 0.0)
    mn = jnp.where(has, mn_ref[...], 0.0)
    aggr = jnp.concatenate([mean, mx, mn, std], axis=-1)
    logd = jnp.log(deg + 1.0)
    big = jnp.concatenate(
        [x_ref[...], aggr, aggr * (logd / AVG_DEG_LOG),
         aggr * (AVG_DEG_LOG / logd)], axis=-1)
    out = _dot(big, wpost_ref[...]) + bpost_ref[...]
    conv = _dot(out, wlin_ref[...]) + blin_ref[...]
    conv_ref[...] = conv
    parts_ref[0, 0, :] = jnp.sum(conv, axis=0)
    parts_ref[0, 1, :] = jnp.sum(conv * conv, axis=0)


def _node(s, sq, mx, mn, cnt, x, Wpost, bpost, Wlin, blin):
    return pl.pallas_call(
        _node_body,
        grid=(GN,),
        in_specs=[
            pl.BlockSpec((NB, D), lambda i: (i, 0)),
            pl.BlockSpec((NB, D), lambda i: (i, 0)),
            pl.BlockSpec((NB, D), lambda i: (i, 0)),
            pl.BlockSpec((NB, D), lambda i: (i, 0)),
            pl.BlockSpec((NB, 1), lambda i: (i, 0)),
            pl.BlockSpec((NB, D), lambda i: (i, 0)),
            pl.BlockSpec((13 * D, D), lambda i: (0, 0)),
            pl.BlockSpec((1, D), lambda i: (0, 0)),
            pl.BlockSpec((D, D), lambda i: (0, 0)),
            pl.BlockSpec((1, D), lambda i: (0, 0)),
        ],
        out_specs=[
            pl.BlockSpec((NB, D), lambda i: (i, 0)),
            pl.BlockSpec((1, 2, D), lambda i: (i, 0, 0)),
        ],
        out_shape=[
            jax.ShapeDtypeStruct((N, D), jnp.float32),
            jax.ShapeDtypeStruct((GN, 2, D), jnp.float32),
        ],
    )(s, sq, mx, mn, cnt, x, Wpost, bpost.reshape(1, D), Wlin,
      blin.reshape(1, D))


def _finish_body(parts_ref, conv_ref, x_ref, gamma_ref, beta_ref,
                 wu1_ref, xn_ref, xnu1_ref, xnu2_ref):
    mu = jnp.sum(parts_ref[:, 0, :], axis=0) * (1.0 / N)
    msq = jnp.sum(parts_ref[:, 1, :], axis=0) * (1.0 / N)
    var = msq - mu * mu
    scale = gamma_ref[0, :] / jnp.sqrt(var + 1e-5)
    bn = scale * (conv_ref[...] - mu) + beta_ref[0, :]
    xn = (x_ref[...] + jnp.maximum(bn, 0.0)) * 0.5
    xn_ref[...] = xn
    u1a = wu1_ref[0:D, :]
    u1b = wu1_ref[D:2 * D, :]
    xnu1_ref[...] = _dot(xn, u1a)
    xnu2_ref[...] = _dot(xn, u1b)


def _finish(parts, conv, x, gamma, beta, Wu1):
    return pl.pallas_call(
        _finish_body,
        grid=(GN,),
        in_specs=[
            pl.BlockSpec((GN, 2, D), lambda i: (0, 0, 0)),
            pl.BlockSpec((NB, D), lambda i: (i, 0)),
            pl.BlockSpec((NB, D), lambda i: (i, 0)),
            pl.BlockSpec((1, D), lambda i: (0, 0)),
            pl.BlockSpec((1, D), lambda i: (0, 0)),
            pl.BlockSpec((3 * D, D), lambda i: (0, 0)),
        ],
        out_specs=[
            pl.BlockSpec((NB, D), lambda i: (i, 0)),
            pl.BlockSpec((NB, D), lambda i: (i, 0)),
            pl.BlockSpec((NB, D), lambda i: (i, 0)),
        ],
        out_shape=[
            jax.ShapeDtypeStruct((N, D), jnp.float32),
            jax.ShapeDtypeStruct((N, D), jnp.float32),
            jax.ShapeDtypeStruct((N, D), jnp.float32),
        ],
    )(parts, conv, x, gamma.reshape(1, D), beta.reshape(1, D), Wu1)


def _edge_upd_body(g2_ref, ea_ref, wu1_ref, bu1_ref, wu2_ref, bu2_ref,
                   en_ref):
    u1c = wu1_ref[2 * D:3 * D, :]
    ea = ea_ref[...]
    eh = jnp.maximum(g2_ref[...] + _dot(ea, u1c) + bu1_ref[...], 0.0)
    eh = _dot(eh, wu2_ref[...]) + bu2_ref[...]
    en_ref[...] = ea + eh * 0.5


def _edge_upd(g2, ea, Wu1, bu1, Wu2, bu2):
    return pl.pallas_call(
        _edge_upd_body,
        grid=(GE,),
        in_specs=[
            pl.BlockSpec((EB, D), lambda i: (i, 0)),
            pl.BlockSpec((EB, D), lambda i: (i, 0)),
            pl.BlockSpec((3 * D, D), lambda i: (0, 0)),
            pl.BlockSpec((1, D), lambda i: (0, 0)),
            pl.BlockSpec((D, D), lambda i: (0, 0)),
            pl.BlockSpec((1, D), lambda i: (0, 0)),
        ],
        out_specs=pl.BlockSpec((EB, D), lambda i: (i, 0)),
        out_shape=jax.ShapeDtypeStruct((E, D), jnp.float32),
    )(g2, ea, Wu1, bu1.reshape(1, D), Wu2, bu2.reshape(1, D))


def kernel(x_gnn, edge_index, edge_attr, We, be, Wpre, bpre, Wpost, bpost,
           Wlin, blin, gamma, beta, Wu1, bu1, Wu2, bu2):
    src = edge_index[0]
    dst = edge_index[1]

    xw1, xw2, Wea, cea = _prep(x_gnn, Wpre, We, be, bpre)

    g = xw1[dst] + xw2[src]
    m = _edge_m(g, edge_attr, Wea, cea)

    cnt = jax.ops.segment_sum(jnp.ones((E,), jnp.float32), dst, N)
    s = jax.ops.segment_sum(m, dst, N)
    sq = jax.ops.segment_sum(m * m, dst, N)
    mx = jax.ops.segment_max(m, dst, N)
    mn = jax.ops.segment_min(m, dst, N)
    has = (cnt > 0)[:, None]
    mx = jnp.where(has, mx, 0.0)
    mn = jnp.where(has, mn, 0.0)

    conv, parts = _node(s, sq, mx, mn, cnt.reshape(N, 1), x_gnn,
                        Wpost, bpost, Wlin, blin)
    x_new, xnu1, xnu2 = _finish(parts, conv, x_gnn, gamma, beta, Wu1)

    g2 = xnu1[src] + xnu2[dst]
    e_new = _edge_upd(g2, edge_attr, Wu1, bu1, Wu2, bu2)
    return x_new, e_new


# speedup vs baseline: 1.5437x; 1.5437x over previous
import functools
import math

import jax
import jax.numpy as jnp
from jax import lax
from jax.experimental import pallas as pl
from jax.experimental.pallas import tpu as pltpu
from jax.experimental.pallas import tpu_sc as plsc

N = 50000
E = 800000
D = 64
AVG_DEG_LOG = math.log(17.0)

NB = 1000
EB = 2000
GN = N // NB
GE = E // EB

_HIGH = jax.lax.Precision.HIGHEST


def _dot(a, b):
    return jax.lax.dot_general(a, b, (((1,), (0,)), ((), ())),
                               precision=_HIGH,
                               preferred_element_type=jnp.float32)


def _prep_body(x_ref, wpre_ref, we_ref, be_ref, bpre_ref,
               xw12_ref, wea_ref, cea_ref):
    w1 = wpre_ref[0:D, :]
    w2 = wpre_ref[D:2 * D, :]
    w3 = wpre_ref[2 * D:3 * D, :]
    w12 = jnp.concatenate([w1, w2], axis=1)
    xw12_ref[...] = _dot(x_ref[...], w12)
    wea_ref[...] = _dot(we_ref[...], w3)
    cea_ref[...] = _dot(be_ref[...], w3) + bpre_ref[...]


def _prep(x, Wpre, We, be, bpre):
    return pl.pallas_call(
        _prep_body,
        grid=(GN,),
        in_specs=[
            pl.BlockSpec((NB, D), lambda i: (i, 0)),
            pl.BlockSpec((3 * D, D), lambda i: (0, 0)),
            pl.BlockSpec((D, D), lambda i: (0, 0)),
            pl.BlockSpec((1, D), lambda i: (0, 0)),
            pl.BlockSpec((1, D), lambda i: (0, 0)),
        ],
        out_specs=[
            pl.BlockSpec((NB, 2 * D), lambda i: (i, 0)),
            pl.BlockSpec((D, D), lambda i: (0, 0)),
            pl.BlockSpec((1, D), lambda i: (0, 0)),
        ],
        out_shape=[
            jax.ShapeDtypeStruct((N, 2 * D), jnp.float32),
            jax.ShapeDtypeStruct((D, D), jnp.float32),
            jax.ShapeDtypeStruct((1, D), jnp.float32),
        ],
    )(x, Wpre, We, be.reshape(1, D), bpre.reshape(1, D))


def _edge_m_body(g_ref, ea_ref, wea_ref, cea_ref, m_ref):
    m_ref[...] = g_ref[...] + _dot(ea_ref[...], wea_ref[...]) + cea_ref[...]


def _edge_m(g, ea, Wea, cea):
    return pl.pallas_call(
        _edge_m_body,
        grid=(GE,),
        in_specs=[
            pl.BlockSpec((EB, D), lambda i: (i, 0)),
            pl.BlockSpec((EB, D), lambda i: (i, 0)),
            pl.BlockSpec((D, D), lambda i: (0, 0)),
            pl.BlockSpec((1, D), lambda i: (0, 0)),
        ],
        out_specs=pl.BlockSpec((EB, D), lambda i: (i, 0)),
        out_shape=jax.ShapeDtypeStruct((E, D), jnp.float32),
    )(g, ea, Wea, cea)


def _node_body(sum_ref, sq_ref, mx_ref, mn_ref, cnt_ref, x_ref,
               wpost_ref, bpost_ref, wlin_ref, blin_ref,
               conv_ref, parts_ref):
    cnt = cnt_ref[...]
    deg = jnp.maximum(cnt, 1.0)
    inv = 1.0 / deg
    mean = sum_ref[...] * inv
    mean2 = sq_ref[...] * inv
    std = jnp.sqrt(jnp.maximum(mean2 - mean * mean, 0.0) + 1e-5)
    has = cnt > 0.0
    mx = jnp.where(has, mx_ref[...], 0.0)
    mn = jnp.where(has, mn_ref[...], 0.0)
    aggr = jnp.concatenate([mean, mx, mn, std], axis=-1)
    logd = jnp.log(deg + 1.0)
    big = jnp.concatenate(
        [x_ref[...], aggr, aggr * (logd / AVG_DEG_LOG),
         aggr * (AVG_DEG_LOG / logd)], axis=-1)
    out = _dot(big, wpost_ref[...]) + bpost_ref[...]
    conv = _dot(out, wlin_ref[...]) + blin_ref[...]
    conv_ref[...] = conv
    parts_ref[0, 0, :] = jnp.sum(conv, axis=0)
    parts_ref[0, 1, :] = jnp.sum(conv * conv, axis=0)


def _node(s, sq, mx, mn, cnt, x, Wpost, bpost, Wlin, blin):
    return pl.pallas_call(
        _node_body,
        grid=(GN,),
        in_specs=[
            pl.BlockSpec((NB, D), lambda i: (i, 0)),
            pl.BlockSpec((NB, D), lambda i: (i, 0)),
            pl.BlockSpec((NB, D), lambda i: (i, 0)),
            pl.BlockSpec((NB, D), lambda i: (i, 0)),
            pl.BlockSpec((NB, 1), lambda i: (i, 0)),
            pl.BlockSpec((NB, D), lambda i: (i, 0)),
            pl.BlockSpec((13 * D, D), lambda i: (0, 0)),
            pl.BlockSpec((1, D), lambda i: (0, 0)),
            pl.BlockSpec((D, D), lambda i: (0, 0)),
            pl.BlockSpec((1, D), lambda i: (0, 0)),
        ],
        out_specs=[
            pl.BlockSpec((NB, D), lambda i: (i, 0)),
            pl.BlockSpec((1, 2, D), lambda i: (i, 0, 0)),
        ],
        out_shape=[
            jax.ShapeDtypeStruct((N, D), jnp.float32),
            jax.ShapeDtypeStruct((GN, 2, D), jnp.float32),
        ],
    )(s, sq, mx, mn, cnt, x, Wpost, bpost.reshape(1, D), Wlin,
      blin.reshape(1, D))


def _finish_body(parts_ref, conv_ref, x_ref, gamma_ref, beta_ref,
                 wu1_ref, xn_ref, xnu_ref):
    mu = jnp.sum(parts_ref[:, 0, :], axis=0) * (1.0 / N)
    msq = jnp.sum(parts_ref[:, 1, :], axis=0) * (1.0 / N)
    var = msq - mu * mu
    scale = gamma_ref[0, :] / jnp.sqrt(var + 1e-5)
    bn = scale * (conv_ref[...] - mu) + beta_ref[0, :]
    xn = (x_ref[...] + jnp.maximum(bn, 0.0)) * 0.5
    xn_ref[...] = xn
    u1ab = jnp.concatenate([wu1_ref[0:D, :], wu1_ref[D:2 * D, :]], axis=1)
    xnu_ref[...] = _dot(xn, u1ab)


def _finish(parts, conv, x, gamma, beta, Wu1):
    return pl.pallas_call(
        _finish_body,
        grid=(GN,),
        in_specs=[
            pl.BlockSpec((GN, 2, D), lambda i: (0, 0, 0)),
            pl.BlockSpec((NB, D), lambda i: (i, 0)),
            pl.BlockSpec((NB, D), lambda i: (i, 0)),
            pl.BlockSpec((1, D), lambda i: (0, 0)),
            pl.BlockSpec((1, D), lambda i: (0, 0)),
            pl.BlockSpec((3 * D, D), lambda i: (0, 0)),
        ],
        out_specs=[
            pl.BlockSpec((NB, D), lambda i: (i, 0)),
            pl.BlockSpec((NB, 2 * D), lambda i: (i, 0)),
        ],
        out_shape=[
            jax.ShapeDtypeStruct((N, D), jnp.float32),
            jax.ShapeDtypeStruct((N, 2 * D), jnp.float32),
        ],
    )(parts, conv, x, gamma.reshape(1, D), beta.reshape(1, D), Wu1)


def _edge_upd_body(g2_ref, ea_ref, wu1_ref, bu1_ref, wu2_ref, bu2_ref,
                   en_ref):
    u1c = wu1_ref[2 * D:3 * D, :]
    ea = ea_ref[...]
    eh = jnp.maximum(g2_ref[...] + _dot(ea, u1c) + bu1_ref[...], 0.0)
    eh = _dot(eh, wu2_ref[...]) + bu2_ref[...]
    en_ref[...] = ea + eh * 0.5


def _edge_upd(g2, ea, Wu1, bu1, Wu2, bu2):
    return pl.pallas_call(
        _edge_upd_body,
        grid=(GE,),
        in_specs=[
            pl.BlockSpec((EB, D), lambda i: (i, 0)),
            pl.BlockSpec((EB, D), lambda i: (i, 0)),
            pl.BlockSpec((3 * D, D), lambda i: (0, 0)),
            pl.BlockSpec((1, D), lambda i: (0, 0)),
            pl.BlockSpec((D, D), lambda i: (0, 0)),
            pl.BlockSpec((1, D), lambda i: (0, 0)),
        ],
        out_specs=pl.BlockSpec((EB, D), lambda i: (i, 0)),
        out_shape=jax.ShapeDtypeStruct((E, D), jnp.float32),
    )(g2, ea, Wu1, bu1.reshape(1, D), Wu2, bu2.reshape(1, D))


_NSC = 2
_NSUB = 16
_NW = _NSC * _NSUB
_EW = E // _NW
_GW = 200
_NWIN = _EW // _GW


def _sc_gadd_body(t_hbm, i1_hbm, i2_hbm, out_hbm,
                  i1_v, i2_v, a_v, b_v, o_v, sem1, sem2):
    wid = lax.axis_index("s") * _NSC + lax.axis_index("c")
    base = wid * _EW

    def window(w, _):
        off = base + w * _GW
        pltpu.sync_copy(i1_hbm.at[pl.ds(off, _GW)], i1_v)
        pltpu.sync_copy(i2_hbm.at[pl.ds(off, _GW)], i2_v)
        cp1 = pltpu.async_copy(t_hbm.at[i1_v], a_v, sem1)
        cp2 = pltpu.async_copy(t_hbm.at[i2_v], b_v, sem2)
        cp1.wait()
        cp2.wait()

        def row(i, _):
            for q in range(4):
                sl = pl.ds(q * 16, 16)
                sl2 = pl.ds(D + q * 16, 16)
                o_v[i, sl] = a_v[i, sl] + b_v[i, sl2]
            return 0

        lax.fori_loop(0, _GW, row, 0)
        pltpu.sync_copy(o_v, out_hbm.at[pl.ds(off, _GW), :])
        return 0

    lax.fori_loop(0, _NWIN, window, 0)


def _sc_gadd(t, i1, i2):
    mesh = plsc.VectorSubcoreMesh(core_axis_name="c", subcore_axis_name="s")
    return pl.kernel(
        _sc_gadd_body,
        out_type=jax.ShapeDtypeStruct((E, D), jnp.float32),
        mesh=mesh,
        scratch_types=[
            pltpu.VMEM((_GW,), jnp.int32),
            pltpu.VMEM((_GW,), jnp.int32),
            pltpu.VMEM((_GW, 2 * D), jnp.float32),
            pltpu.VMEM((_GW, 2 * D), jnp.float32),
            pltpu.VMEM((_GW, D), jnp.float32),
            pltpu.SemaphoreType.DMA,
            pltpu.SemaphoreType.DMA,
        ],
    )(t, i1, i2)


def kernel(x_gnn, edge_index, edge_attr, We, be, Wpre, bpre, Wpost, bpost,
           Wlin, blin, gamma, beta, Wu1, bu1, Wu2, bu2):
    src = edge_index[0]
    dst = edge_index[1]

    xw12, Wea, cea = _prep(x_gnn, Wpre, We, be, bpre)

    g = _sc_gadd(xw12, dst, src)
    m = _edge_m(g, edge_attr, Wea, cea)

    cnt = jax.ops.segment_sum(jnp.ones((E,), jnp.float32), dst, N)
    s = jax.ops.segment_sum(m, dst, N)
    sq = jax.ops.segment_sum(m * m, dst, N)
    mx = jax.ops.segment_max(m, dst, N)
    mn = jax.ops.segment_min(m, dst, N)
    has = (cnt > 0)[:, None]
    mx = jnp.where(has, mx, 0.0)
    mn = jnp.where(has, mn, 0.0)

    conv, parts = _node(s, sq, mx, mn, cnt.reshape(N, 1), x_gnn,
                        Wpost, bpost, Wlin, blin)
    x_new, xnu = _finish(parts, conv, x_gnn, gamma, beta, Wu1)

    g2 = _sc_gadd(xnu, src, dst)
    e_new = _edge_upd(g2, edge_attr, Wu1, bu1, Wu2, bu2)
    return x_new, e_new
